# baseline (device time: 80121 ns/iter reference)
import jax
import jax.numpy as jnp
from jax import lax
from jax.experimental import pallas as pl
from jax.experimental.pallas import tpu as pltpu

N_DEV = 8
SQ = 2048
D = 1024
HQ = 8
DH = 128
WIN = 128
CHUNK = 256
N_CHUNK = SQ // CHUNK
BAND = CHUNK + 2 * WIN
EDGE = 256
KTOT = SQ + EDGE
SCALE = 0.08838834764831843

TREE_CHILDREN = {0: (4, 3, 1), 4: (7, 5), 3: (2,), 7: (6,)}
MAX_FANOUT = 3


def kernel(x, Wq, K_ext, V_ext, Wo):
    def body(x_ref, wq_ref, k_ref, v_ref, wo_ref, out_ref,
             kall, vall, stage, estage, edge,
             lsem, esend, erecv, ssend, srecv):
        my = lax.axis_index("i")

        def edge_rdma(dev):
            return pltpu.make_async_remote_copy(
                src_ref=edge, dst_ref=edge,
                send_sem=esend, recv_sem=erecv,
                device_id=(dev,), device_id_type=pl.DeviceIdType.MESH,
            )

        def chunk_rdma(c, j, dev):
            sl = (0, pl.ds(c * CHUNK, CHUNK), slice(None))
            return pltpu.make_async_remote_copy(
                src_ref=out_ref.at[sl], dst_ref=out_ref.at[sl],
                send_sem=ssend.at[c, j], recv_sem=srecv.at[c],
                device_id=(dev,), device_id_type=pl.DeviceIdType.MESH,
            )

        @pl.when(my == 1)
        def _():
            cpk = pltpu.make_async_copy(
                k_ref.at[0, pl.ds(0, EDGE)], estage.at[0], lsem.at[0])
            cpv = pltpu.make_async_copy(
                v_ref.at[0, pl.ds(0, EDGE)], estage.at[1], lsem.at[1])
            cpk.start()
            cpv.start()
            cpk.wait()
            cpv.wait()
            edge[0] = estage[0].reshape(EDGE, D).astype(jnp.bfloat16)
            edge[1] = estage[1].reshape(EDGE, D).astype(jnp.bfloat16)
            snd = edge_rdma(0)
            snd.start()
            snd.wait_send()

        @pl.when(my == 0)
        def _():
            cpk = pltpu.make_async_copy(k_ref.at[0], stage.at[0], lsem.at[0])
            cpv = pltpu.make_async_copy(v_ref.at[0], stage.at[1], lsem.at[1])
            cpk.start()
            cpv.start()
            cpk.wait()
            kall[pl.ds(0, SQ), :] = stage[0].reshape(SQ, D).astype(jnp.bfloat16)
            cpv.wait()
            vall[pl.ds(0, SQ), :] = stage[1].reshape(SQ, D).astype(jnp.bfloat16)

            wq = wq_ref[...].astype(jnp.bfloat16)
            wo = wo_ref[...].astype(jnp.bfloat16)

            r_i = lax.broadcasted_iota(jnp.int32, (CHUNK, BAND), 0)
            col_i = lax.broadcasted_iota(jnp.int32, (CHUNK, BAND), 1)
            neg_mid = jnp.where(
                jnp.abs(r_i - col_i + WIN) <= WIN, 0.0, -1e9
            ).astype(jnp.float32)
            neg_zero = jnp.where(
                jnp.abs(r_i - col_i) <= WIN, 0.0, -1e9
            ).astype(jnp.float32)

            sends = []
            for c in range(N_CHUNK):
                o = max(0, c * CHUNK - WIN)
                if o + BAND > SQ:
                    rcv = edge_rdma(1)
                    rcv.wait_recv()
                    kall[pl.ds(SQ, EDGE), :] = edge[0]
                    vall[pl.ds(SQ, EDGE), :] = edge[1]
                neg = neg_zero if c == 0 else neg_mid
                xc = x_ref[0, pl.ds(c * CHUNK, CHUNK), :].astype(jnp.bfloat16)
                q = jnp.dot(xc, wq, preferred_element_type=jnp.float32)
                q = (q * SCALE).astype(jnp.bfloat16)
                kb = kall[pl.ds(o, BAND), :]
                vb = vall[pl.ds(o, BAND), :]
                ctx_cols = []
                for h in range(HQ):
                    qh = q[:, h * DH:(h + 1) * DH]
                    kh = kb[:, h * DH:(h + 1) * DH]
                    s = lax.dot_general(
                        qh, kh, (((1,), (1,)), ((), ())),
                        preferred_element_type=jnp.float32,
                    ) + neg
                    e = jnp.exp(s)
                    rs = 1.0 / jnp.sum(e, axis=1, keepdims=True)
                    vh = vb[:, h * DH:(h + 1) * DH]
                    ctx_un = jnp.dot(e.astype(jnp.bfloat16), vh,
                                     preferred_element_type=jnp.float32)
                    ctx_cols.append((ctx_un * rs).astype(jnp.bfloat16))
                ctx = jnp.concatenate(ctx_cols, axis=1)
                outc = jnp.dot(ctx, wo, preferred_element_type=jnp.float32)
                out_ref[0, pl.ds(c * CHUNK, CHUNK), :] = outc.astype(jnp.bfloat16)
                for j, child in enumerate(TREE_CHILDREN[0]):
                    snd = chunk_rdma(c, j, child)
                    snd.start()
                    sends.append(snd)
            for snd in sends:
                snd.wait_send()

        for dev in range(1, N_DEV):

            @pl.when(my == dev)
            def _(dev=dev):
                sends = []
                for c in range(N_CHUNK):
                    chunk_rdma(c, 0, 0).wait_recv()
                    for j, child in enumerate(TREE_CHILDREN.get(dev, ())):
                        snd = chunk_rdma(c, j, child)
                        snd.start()
                        sends.append(snd)
                for snd in sends:
                    snd.wait_send()

        return

    return pl.pallas_call(
        body,
        out_shape=jax.ShapeDtypeStruct((1, SQ, D), jnp.bfloat16),
        in_specs=[
            pl.BlockSpec(memory_space=pltpu.VMEM),
            pl.BlockSpec(memory_space=pltpu.VMEM),
            pl.BlockSpec(memory_space=pltpu.MemorySpace.HBM),
            pl.BlockSpec(memory_space=pltpu.MemorySpace.HBM),
            pl.BlockSpec(memory_space=pltpu.VMEM),
        ],
        out_specs=pl.BlockSpec(memory_space=pltpu.VMEM),
        scratch_shapes=[
            pltpu.VMEM((KTOT, D), jnp.bfloat16),
            pltpu.VMEM((KTOT, D), jnp.bfloat16),
            pltpu.VMEM((2, SQ, HQ, DH), jnp.float32),
            pltpu.VMEM((2, EDGE, HQ, DH), jnp.float32),
            pltpu.VMEM((2, EDGE, D), jnp.bfloat16),
            pltpu.SemaphoreType.DMA((2,)),
            pltpu.SemaphoreType.DMA,
            pltpu.SemaphoreType.DMA,
            pltpu.SemaphoreType.DMA((N_CHUNK, MAX_FANOUT)),
            pltpu.SemaphoreType.DMA((N_CHUNK,)),
        ],
        compiler_params=pltpu.CompilerParams(
            vmem_limit_bytes=100 * 1024 * 1024,
        ),
    )(x, Wq, K_ext, V_ext, Wo)
